# baseline (speedup 1.0000x reference)
import functools

import jax
import jax.numpy as jnp
from jax.experimental import pallas as pl
import jax.experimental.pallas.tpu as pltpu


def _layer1_body(x_ref, w1_ref, b1_ref, adj_ref, x1_ref, s1_ref):
    i = pl.program_id(0)

    @pl.when(i == 0)
    def _():
        s1_ref[...] = jnp.dot(x_ref[...], w1_ref[...],
                              preferred_element_type=jnp.float32)

    y = jnp.dot(adj_ref[...], s1_ref[...],
                preferred_element_type=jnp.float32)
    x1_ref[...] = jnp.maximum(y + b1_ref[...], 0.0)


def _layer2_body(x1_ref, w2_ref, b2_ref, adj_ref, out_ref, s2_ref):
    i = pl.program_id(0)

    @pl.when(i == 0)
    def _():
        s2_ref[...] = jnp.dot(x1_ref[...], w2_ref[...],
                              preferred_element_type=jnp.float32)

    y = jnp.dot(adj_ref[...], s2_ref[...],
                preferred_element_type=jnp.float32) + b2_ref[...]
    m = jnp.max(y, axis=1, keepdims=True)
    z = y - m
    lse = jnp.log(jnp.sum(jnp.exp(z), axis=1, keepdims=True))
    out_ref[...] = z - lse


@functools.partial(jax.jit, static_argnames=("block_rows",))
def _gcn(x, adj, W1, b1, W2, b2, block_rows=400):
    n, d_in = x.shape
    d_hid = W1.shape[1]
    d_out = W2.shape[1]
    grid = (n // block_rows,)

    full = lambda s: pl.BlockSpec(s, lambda i: (0, 0))
    rows = lambda c: pl.BlockSpec((block_rows, c), lambda i: (i, 0))
    adj_spec = pl.BlockSpec((block_rows, n), lambda i: (i, 0))

    x1 = pl.pallas_call(
        _layer1_body,
        grid=grid,
        in_specs=[full((n, d_in)), full((d_in, d_hid)), full((1, d_hid)),
                  adj_spec],
        out_specs=rows(d_hid),
        out_shape=jax.ShapeDtypeStruct((n, d_hid), jnp.float32),
        scratch_shapes=[pltpu.VMEM((n, d_hid), jnp.float32)],
    )(x, W1, b1.reshape(1, d_hid), adj)

    out = pl.pallas_call(
        _layer2_body,
        grid=grid,
        in_specs=[full((n, d_hid)), full((d_hid, d_out)), full((1, d_out)),
                  adj_spec],
        out_specs=rows(d_out),
        out_shape=jax.ShapeDtypeStruct((n, d_out), jnp.float32),
        scratch_shapes=[pltpu.VMEM((n, d_out), jnp.float32)],
    )(x1, W2, b2.reshape(1, d_out), adj)

    return out, x1


def kernel(x, adj, W1, b1, W2, b2):
    return _gcn(x, adj, W1, b1, W2, b2)
